# baseline (speedup 1.0000x reference)
import functools

import jax
import jax.numpy as jnp
from jax import lax
from jax.experimental import pallas as pl
from jax.experimental.pallas import tpu as pltpu
from jax.experimental.pallas import tpu_sc as plsc

N = 10000
E = 320000
D_IN = 128
D_H = 256
D_OUT = 128
DHALF = 64

NC = 2
NS = 16
CE = 128
CPW = 160
NBUF = 5
LOOK = 2
E_PAD = NS * CPW * CE
PAD_ROWS = 112
NP = N + PAD_ROWS
RT = NP // NS
ZB = 80

ROWS_B = 10
RB = N // ROWS_B

_mesh = plsc.VectorSubcoreMesh(
    core_axis_name="c", subcore_axis_name="s", num_cores=NC, num_subcores=NS)
_sc_params = pltpu.CompilerParams(use_tc_tiling_on_sc=False)


def _zero_vmem_f32(ref, nrows, ncols):
    zv = jnp.zeros((16,), jnp.float32)

    def body(i, _):
        for c in range(ncols // 16):
            ref[i, pl.ds(c * 16, 16)] = zv
        return 0

    lax.fori_loop(0, nrows, body, 0)


def _deg_body(dstr_hbm, out_hbm, dst_v, ones_v, z16_v, dacc, sem):
    cid = lax.axis_index("c")
    sid = lax.axis_index("s")
    pltpu.async_copy(dstr_hbm.at[sid], dst_v, sem).wait()

    ov = jnp.ones((16,), jnp.float32)

    def fill_ones(i, _):
        ones_v[i, pl.ds(0, 16)] = ov
        return 0

    lax.fori_loop(0, CE, fill_ones, 0)
    _zero_vmem_f32(z16_v, RT, 16)
    base = sid * RT
    pltpu.sync_copy(z16_v, dacc.at[pl.ds(base, RT)])
    plsc.subcore_barrier()

    half = CPW // 2
    lo = cid * half

    def body(j, _):
        pltpu.sync_copy(ones_v, dacc.at[dst_v.at[lo + j]], add=True)
        return 0

    lax.fori_loop(0, half, body, 0)
    plsc.subcore_barrier()
    pltpu.sync_copy(dacc.at[pl.ds(base, RT)], out_hbm.at[cid, pl.ds(base, RT)])


@functools.partial(
    pl.kernel,
    out_type=jax.ShapeDtypeStruct((NC, NP, 16), jnp.float32),
    mesh=_mesh,
    compiler_params=_sc_params,
    scratch_types=[
        pltpu.VMEM((CPW, CE), jnp.int32),
        pltpu.VMEM((CE, 16), jnp.float32),
        pltpu.VMEM((RT, 16), jnp.float32),
        pltpu.VMEM_SHARED((NP, 16), jnp.float32),
        pltpu.SemaphoreType.DMA,
    ],
)
def _deg_kernel(dstr_hbm, out_hbm, dst_v, ones_v, z16_v, dacc, sem):
    _deg_body(dstr_hbm, out_hbm, dst_v, ones_v, z16_v, dacc, sem)


def _zero_acc_slice(rows_v, acc, base):
    zv = jnp.zeros((16,), jnp.float32)

    def zbody(i, _):
        for c in range(DHALF // 16):
            rows_v[0, i, pl.ds(c * 16, 16)] = zv
        return 0

    lax.fori_loop(0, CE, zbody, 0)
    for t in range(4):
        pltpu.sync_copy(rows_v.at[0], acc.at[pl.ds(base + t * CE, CE)])
    pltpu.sync_copy(rows_v.at[0, pl.ds(0, RT - 4 * CE)],
                    acc.at[pl.ds(base + 4 * CE, RT - 4 * CE)])


def _edge_pipeline(v_hbm, src_v, dst_v, rows, acc, gsem, ssem):
    for b in range(LOOK):
        pltpu.async_copy(v_hbm.at[src_v.at[b]], rows[b], gsem[b])

    def body(j, _):
        for b in range(NBUF):
            c = NBUF * j + b
            pltpu.make_async_copy(
                v_hbm.at[src_v.at[c]], rows[b], gsem[b]).wait()
            pltpu.async_copy(rows[b], acc.at[dst_v.at[c]], ssem[b],
                             add=True)
            cg = c + LOOK
            bg = (b + LOOK) % NBUF

            @pl.when(cg < CPW)
            def _():
                @pl.when(cg >= NBUF)
                def _():
                    pltpu.make_async_copy(
                        rows[bg], acc.at[dst_v.at[cg - NBUF]],
                        ssem[bg]).wait()

                pltpu.async_copy(
                    v_hbm.at[src_v.at[cg]], rows[bg], gsem[bg])

        return 0

    lax.fori_loop(0, CPW // NBUF, body, 0)
    for b in range(NBUF):
        pltpu.make_async_copy(
            rows[b], acc.at[dst_v.at[CPW - NBUF + b]], ssem[b]).wait()


def _agg_body(vl_hbm, vh_hbm, srcr_hbm, dstr_hbm, out_hbm,
              src_v, dst_v, rows_v, acc, gsems, ssems):
    rows = [rows_v.at[b] for b in range(NBUF)]
    gsem = [gsems.at[b] for b in range(NBUF)]
    ssem = [ssems.at[b] for b in range(NBUF)]
    cid = lax.axis_index("c")
    sid = lax.axis_index("s")
    pltpu.async_copy(srcr_hbm.at[sid], src_v, gsem[0]).wait()
    pltpu.async_copy(dstr_hbm.at[sid], dst_v, gsem[0]).wait()

    base = sid * RT
    _zero_acc_slice(rows_v, acc, base)
    plsc.subcore_barrier()

    @pl.when(cid == 0)
    def _():
        _edge_pipeline(vl_hbm, src_v, dst_v, rows, acc, gsem, ssem)

    @pl.when(cid == 1)
    def _():
        _edge_pipeline(vh_hbm, src_v, dst_v, rows, acc, gsem, ssem)

    plsc.subcore_barrier()
    pltpu.sync_copy(acc.at[pl.ds(base, RT)], out_hbm.at[cid, pl.ds(base, RT)])


@functools.partial(
    pl.kernel,
    out_type=jax.ShapeDtypeStruct((NC, NP, DHALF), jnp.float32),
    mesh=_mesh,
    compiler_params=_sc_params,
    scratch_types=[
        pltpu.VMEM((CPW, CE), jnp.int32),
        pltpu.VMEM((CPW, CE), jnp.int32),
        pltpu.VMEM((NBUF, CE, DHALF), jnp.float32),
        pltpu.VMEM_SHARED((NP, DHALF), jnp.float32),
        pltpu.SemaphoreType.DMA((NBUF,)),
        pltpu.SemaphoreType.DMA((NBUF,)),
    ],
)
def _agg_kernel(vl_hbm, vh_hbm, srcr_hbm, dstr_hbm, out_hbm,
                src_v, dst_v, rows_v, acc, gsems, ssems):
    _agg_body(vl_hbm, vh_hbm, srcr_hbm, dstr_hbm, out_hbm,
              src_v, dst_v, rows_v, acc, gsems, ssems)


CROWS = N // NS
CTAIL = CROWS - 4 * CE

def _agg_combine_body(vl_hbm, vh_hbm, srcr_hbm, dstr_hbm, dinv_hbm, b2_hbm,
                      out_hbm, src_v, dst_v, rows_v, cd_v, b2_v,
                      acc, gsems, ssems):
    rows = [rows_v.at[b] for b in range(NBUF)]
    gsem = [gsems.at[b] for b in range(NBUF)]
    ssem = [ssems.at[b] for b in range(NBUF)]
    cid = lax.axis_index("c")
    sid = lax.axis_index("s")
    pltpu.async_copy(srcr_hbm.at[sid], src_v, gsem[0]).wait()
    pltpu.async_copy(dstr_hbm.at[sid], dst_v, gsem[0]).wait()
    pltpu.async_copy(b2_hbm, b2_v, gsem[1]).wait()

    base = sid * RT
    _zero_acc_slice(rows_v, acc, base)
    plsc.subcore_barrier()

    @pl.when(cid == 0)
    def _():
        _edge_pipeline(vl_hbm, src_v, dst_v, rows, acc, gsem, ssem)

    @pl.when(cid == 1)
    def _():
        _edge_pipeline(vh_hbm, src_v, dst_v, rows, acc, gsem, ssem)

    plsc.subcore_barrier()

    b2c = [b2_v[cid * 4 + c, pl.ds(0, 16)] for c in range(4)]
    cbase = sid * CROWS

    def combine(v_hbm):
        for t in range(5):
            nn = CE if t < 4 else CTAIL
            r0 = cbase + t * CE
            pltpu.sync_copy(acc.at[pl.ds(r0, nn)],
                            rows_v.at[0, pl.ds(0, nn)])
            pltpu.sync_copy(v_hbm.at[pl.ds(r0, nn)],
                            rows_v.at[1, pl.ds(0, nn)])
            pltpu.sync_copy(dinv_hbm.at[pl.ds(r0, nn)],
                            cd_v.at[pl.ds(0, nn)])

            def cb(i, _):
                dv = cd_v[i, pl.ds(0, 16)]
                for c in range(4):
                    a = rows_v[0, i, pl.ds(c * 16, 16)]
                    z = rows_v[1, i, pl.ds(c * 16, 16)]
                    rows_v[2, i, pl.ds(c * 16, 16)] = (a + z) * dv + b2c[c]
                return 0

            lax.fori_loop(0, nn, cb, 0)
            pltpu.sync_copy(
                rows_v.at[2, pl.ds(0, nn)],
                out_hbm.at[pl.ds(r0, nn), pl.ds(cid * DHALF, DHALF)])

    @pl.when(cid == 0)
    def _():
        combine(vl_hbm)

    @pl.when(cid == 1)
    def _():
        combine(vh_hbm)


@functools.partial(
    pl.kernel,
    out_type=jax.ShapeDtypeStruct((N, D_OUT), jnp.float32),
    mesh=_mesh,
    compiler_params=_sc_params,
    scratch_types=[
        pltpu.VMEM((CPW, CE), jnp.int32),
        pltpu.VMEM((CPW, CE), jnp.int32),
        pltpu.VMEM((NBUF, CE, DHALF), jnp.float32),
        pltpu.VMEM((CE, 16), jnp.float32),
        pltpu.VMEM((8, 16), jnp.float32),
        pltpu.VMEM_SHARED((NP, DHALF), jnp.float32),
        pltpu.SemaphoreType.DMA((NBUF,)),
        pltpu.SemaphoreType.DMA((NBUF,)),
    ],
)
def _agg_combine_kernel(vl_hbm, vh_hbm, srcr_hbm, dstr_hbm, dinv_hbm, b2_hbm,
                        out_hbm, src_v, dst_v, rows_v, cd_v, b2_v,
                        acc, gsems, ssems):
    _agg_combine_body(vl_hbm, vh_hbm, srcr_hbm, dstr_hbm, dinv_hbm, b2_hbm,
                      out_hbm, src_v, dst_v, rows_v, cd_v, b2_v,
                      acc, gsems, ssems)


def _dinv_block(degp_ref):
    deg = degp_ref[0, :, 0:1] + degp_ref[1, :, 0:1] + 1.0
    return lax.rsqrt(deg)


def _scale_x_body(degp_ref, x_ref, xsl_ref, xsh_ref):
    xs = x_ref[...] * _dinv_block(degp_ref)
    xsl_ref[...] = xs[:, :DHALF]
    xsh_ref[...] = xs[:, DHALF:]


def _mlp_body(degp_ref, sp_ref, xsl_ref, xsh_ref, w1_ref, b1_ref, w2_ref,
              z2sl_ref, z2sh_ref, dinv16_ref):
    dinv = _dinv_block(degp_ref)
    aggx = jnp.concatenate(
        [sp_ref[0] + xsl_ref[...], sp_ref[1] + xsh_ref[...]], axis=1) * dinv
    h1 = jnp.dot(aggx, w1_ref[...], preferred_element_type=jnp.float32)
    h1 = jnp.maximum(h1 + b1_ref[...], 0.0)
    z2 = jnp.dot(h1, w2_ref[...], preferred_element_type=jnp.float32)
    z2s = z2 * dinv
    z2sl_ref[...] = z2s[:, :DHALF]
    z2sh_ref[...] = z2s[:, DHALF:]
    dinv16_ref[...] = jnp.broadcast_to(dinv, (RB, 16))


def _out_body(degp_ref, sp2_ref, z2sl_ref, z2sh_ref, b2_ref, out_ref):
    dinv = _dinv_block(degp_ref)
    s = jnp.concatenate(
        [sp2_ref[0] + z2sl_ref[...], sp2_ref[1] + z2sh_ref[...]], axis=1)
    out_ref[...] = s * dinv + b2_ref[...]


def _degp_spec():
    return pl.BlockSpec((2, RB, 16), lambda r: (0, r, 0))


def _rows_spec(width):
    return pl.BlockSpec((RB, width), lambda r: (r, 0))


def _part_spec():
    return pl.BlockSpec((2, RB, DHALF), lambda r: (0, r, 0))


def _full_spec(shape):
    nd = len(shape)
    return pl.BlockSpec(shape, lambda r: (0,) * nd)


def _scale_x(degp, x):
    return pl.pallas_call(
        _scale_x_body,
        grid=(ROWS_B,),
        in_specs=[_degp_spec(), _rows_spec(128)],
        out_specs=(_rows_spec(DHALF), _rows_spec(DHALF)),
        out_shape=(jax.ShapeDtypeStruct((N, DHALF), jnp.float32),
                   jax.ShapeDtypeStruct((N, DHALF), jnp.float32)),
    )(degp, x)


def _mlp(degp, sp, xsl, xsh, w1, b1, w2):
    return pl.pallas_call(
        _mlp_body,
        grid=(ROWS_B,),
        in_specs=[_degp_spec(), _part_spec(), _rows_spec(DHALF),
                  _rows_spec(DHALF),
                  _full_spec((D_IN, D_H)), _full_spec((1, D_H)),
                  _full_spec((D_H, D_OUT))],
        out_specs=(_rows_spec(DHALF), _rows_spec(DHALF), _rows_spec(16)),
        out_shape=(jax.ShapeDtypeStruct((N, DHALF), jnp.float32),
                   jax.ShapeDtypeStruct((N, DHALF), jnp.float32),
                   jax.ShapeDtypeStruct((N, 16), jnp.float32)),
    )(degp, sp, xsl, xsh, w1, b1, w2)


def _combine_out(degp, sp2, z2sl, z2sh, b2):
    return pl.pallas_call(
        _out_body,
        grid=(ROWS_B,),
        in_specs=[_degp_spec(), _part_spec(), _rows_spec(DHALF),
                  _rows_spec(DHALF), _full_spec((1, D_OUT))],
        out_specs=_rows_spec(128),
        out_shape=jax.ShapeDtypeStruct((N, D_OUT), jnp.float32),
    )(degp, sp2, z2sl, z2sh, b2)


def kernel(x, edge_index, W1, b1, W2, b2):
    pad = E_PAD - E
    apad = jnp.arange(pad, dtype=jnp.int32)
    src_p = jnp.concatenate([edge_index[0], apad % N])
    dst_p = jnp.concatenate([edge_index[1], N + apad % PAD_ROWS])
    srcr = src_p.reshape(NS, CPW, CE)
    dstr = dst_p.reshape(NS, CPW, CE)

    degp = _deg_kernel(dstr)
    xsl, xsh = _scale_x(degp, x)
    sp = _agg_kernel(xsl, xsh, srcr, dstr)
    z2sl, z2sh, dinv16 = _mlp(degp, sp, xsl, xsh, W1, b1.reshape(1, D_H), W2)
    return _agg_combine_kernel(z2sl, z2sh, srcr, dstr, dinv16,
                               b2.reshape(8, 16))


# speedup vs baseline: 1.0328x; 1.0328x over previous
import functools

import jax
import jax.numpy as jnp
from jax import lax
from jax.experimental import pallas as pl
from jax.experimental.pallas import tpu as pltpu
from jax.experimental.pallas import tpu_sc as plsc

N = 10000
E = 320000
D_IN = 128
D_H = 256
D_OUT = 128
DHALF = 64

NC = 2
NS = 16
CE = 128
CPW = 160
NBUF1 = 6
NBUF2 = 5
E_PAD = NS * CPW * CE
PAD_ROWS = 112
NP = N + PAD_ROWS
RT = NP // NS
ZB = 80

ROWS_B = 10
RB = N // ROWS_B

_mesh = plsc.VectorSubcoreMesh(
    core_axis_name="c", subcore_axis_name="s", num_cores=NC, num_subcores=NS)
_sc_params = pltpu.CompilerParams(use_tc_tiling_on_sc=False)


def _zero_vmem_f32(ref, nrows, ncols):
    zv = jnp.zeros((16,), jnp.float32)

    def body(i, _):
        for c in range(ncols // 16):
            ref[i, pl.ds(c * 16, 16)] = zv
        return 0

    lax.fori_loop(0, nrows, body, 0)


def _deg_body(dstr_hbm, out_hbm, dst_v, ones_v, z16_v, dacc, sem):
    cid = lax.axis_index("c")
    sid = lax.axis_index("s")
    pltpu.async_copy(dstr_hbm.at[sid], dst_v, sem).wait()

    ov = jnp.ones((16,), jnp.float32)

    def fill_ones(i, _):
        ones_v[i, pl.ds(0, 16)] = ov
        return 0

    lax.fori_loop(0, CE, fill_ones, 0)
    _zero_vmem_f32(z16_v, RT, 16)
    base = sid * RT
    pltpu.sync_copy(z16_v, dacc.at[pl.ds(base, RT)])
    plsc.subcore_barrier()

    half = CPW // 2
    lo = cid * half

    def body(j, _):
        pltpu.sync_copy(ones_v, dacc.at[dst_v.at[lo + j]], add=True)
        return 0

    lax.fori_loop(0, half, body, 0)
    plsc.subcore_barrier()
    pltpu.sync_copy(dacc.at[pl.ds(base, RT)], out_hbm.at[cid, pl.ds(base, RT)])


@functools.partial(
    pl.kernel,
    out_type=jax.ShapeDtypeStruct((NC, NP, 16), jnp.float32),
    mesh=_mesh,
    compiler_params=_sc_params,
    scratch_types=[
        pltpu.VMEM((CPW, CE), jnp.int32),
        pltpu.VMEM((CE, 16), jnp.float32),
        pltpu.VMEM((RT, 16), jnp.float32),
        pltpu.VMEM_SHARED((NP, 16), jnp.float32),
        pltpu.SemaphoreType.DMA,
    ],
)
def _deg_kernel(dstr_hbm, out_hbm, dst_v, ones_v, z16_v, dacc, sem):
    _deg_body(dstr_hbm, out_hbm, dst_v, ones_v, z16_v, dacc, sem)


def _zero_acc_slice(rows_v, acc, base):
    zv = jnp.zeros((16,), jnp.float32)

    def zbody(i, _):
        for c in range(DHALF // 16):
            rows_v[0, i, pl.ds(c * 16, 16)] = zv
        return 0

    lax.fori_loop(0, CE, zbody, 0)
    for t in range(4):
        pltpu.sync_copy(rows_v.at[0], acc.at[pl.ds(base + t * CE, CE)])
    pltpu.sync_copy(rows_v.at[0, pl.ds(0, RT - 4 * CE)],
                    acc.at[pl.ds(base + 4 * CE, RT - 4 * CE)])


def _edge_pipeline(v_hbm, src_v, dst_v, rows, acc, gsem, ssem):
    nbuf = len(rows)
    look = nbuf // 2
    groups = CPW // nbuf
    for b in range(look):
        pltpu.async_copy(v_hbm.at[src_v.at[b]], rows[b], gsem[b])

    def slot_static(c):
        b = c % nbuf
        pltpu.make_async_copy(
            v_hbm.at[src_v.at[c]], rows[b], gsem[b]).wait()
        pltpu.async_copy(rows[b], acc.at[dst_v.at[c]], ssem[b], add=True)
        cg = c + look
        bg = (b + look) % nbuf
        if cg < CPW:
            if cg >= nbuf:
                pltpu.make_async_copy(
                    rows[bg], acc.at[dst_v.at[cg - nbuf]], ssem[bg]).wait()
            pltpu.async_copy(v_hbm.at[src_v.at[cg]], rows[bg], gsem[bg])

    def body(j, _):
        for b in range(nbuf):
            c = nbuf * j + b
            pltpu.make_async_copy(
                v_hbm.at[src_v.at[c]], rows[b], gsem[b]).wait()
            pltpu.async_copy(rows[b], acc.at[dst_v.at[c]], ssem[b],
                             add=True)
            cg = c + look
            bg = (b + look) % nbuf

            @pl.when(cg < CPW)
            def _():
                @pl.when(cg >= nbuf)
                def _():
                    pltpu.make_async_copy(
                        rows[bg], acc.at[dst_v.at[cg - nbuf]],
                        ssem[bg]).wait()

                pltpu.async_copy(
                    v_hbm.at[src_v.at[cg]], rows[bg], gsem[bg])

        return 0

    lax.fori_loop(0, groups, body, 0)
    for c in range(groups * nbuf, CPW):
        slot_static(c)
    for c in range(CPW - nbuf, CPW):
        b = c % nbuf
        pltpu.make_async_copy(rows[b], acc.at[dst_v.at[c]], ssem[b]).wait()


def _agg_body(vl_hbm, vh_hbm, srcr_hbm, dstr_hbm, out_hbm,
              src_v, dst_v, rows_v, acc, gsems, ssems):
    nbuf = rows_v.shape[0]
    rows = [rows_v.at[b] for b in range(nbuf)]
    gsem = [gsems.at[b] for b in range(nbuf)]
    ssem = [ssems.at[b] for b in range(nbuf)]
    cid = lax.axis_index("c")
    sid = lax.axis_index("s")
    pltpu.async_copy(srcr_hbm.at[sid], src_v, gsem[0]).wait()
    pltpu.async_copy(dstr_hbm.at[sid], dst_v, gsem[0]).wait()

    base = sid * RT
    _zero_acc_slice(rows_v, acc, base)
    plsc.subcore_barrier()

    @pl.when(cid == 0)
    def _():
        _edge_pipeline(vl_hbm, src_v, dst_v, rows, acc, gsem, ssem)

    @pl.when(cid == 1)
    def _():
        _edge_pipeline(vh_hbm, src_v, dst_v, rows, acc, gsem, ssem)

    plsc.subcore_barrier()
    pltpu.sync_copy(acc.at[pl.ds(base, RT)], out_hbm.at[cid, pl.ds(base, RT)])


@functools.partial(
    pl.kernel,
    out_type=jax.ShapeDtypeStruct((NC, NP, DHALF), jnp.float32),
    mesh=_mesh,
    compiler_params=_sc_params,
    scratch_types=[
        pltpu.VMEM((CPW, CE), jnp.int32),
        pltpu.VMEM((CPW, CE), jnp.int32),
        pltpu.VMEM((NBUF1, CE, DHALF), jnp.float32),
        pltpu.VMEM_SHARED((NP, DHALF), jnp.float32),
        pltpu.SemaphoreType.DMA((NBUF1,)),
        pltpu.SemaphoreType.DMA((NBUF1,)),
    ],
)
def _agg_kernel(vl_hbm, vh_hbm, srcr_hbm, dstr_hbm, out_hbm,
                src_v, dst_v, rows_v, acc, gsems, ssems):
    _agg_body(vl_hbm, vh_hbm, srcr_hbm, dstr_hbm, out_hbm,
              src_v, dst_v, rows_v, acc, gsems, ssems)


CROWS = N // NS
CTAIL = CROWS - 4 * CE

def _agg_combine_body(vl_hbm, vh_hbm, srcr_hbm, dstr_hbm, dinv_hbm, b2_hbm,
                      out_hbm, src_v, dst_v, rows_v, cd_v, b2_v,
                      acc, gsems, ssems):
    nbuf = rows_v.shape[0]
    rows = [rows_v.at[b] for b in range(nbuf)]
    gsem = [gsems.at[b] for b in range(nbuf)]
    ssem = [ssems.at[b] for b in range(nbuf)]
    cid = lax.axis_index("c")
    sid = lax.axis_index("s")
    pltpu.async_copy(srcr_hbm.at[sid], src_v, gsem[0]).wait()
    pltpu.async_copy(dstr_hbm.at[sid], dst_v, gsem[0]).wait()
    pltpu.async_copy(b2_hbm, b2_v, gsem[1]).wait()

    base = sid * RT
    _zero_acc_slice(rows_v, acc, base)
    plsc.subcore_barrier()

    @pl.when(cid == 0)
    def _():
        _edge_pipeline(vl_hbm, src_v, dst_v, rows, acc, gsem, ssem)

    @pl.when(cid == 1)
    def _():
        _edge_pipeline(vh_hbm, src_v, dst_v, rows, acc, gsem, ssem)

    plsc.subcore_barrier()

    b2c = [b2_v[cid * 4 + c, pl.ds(0, 16)] for c in range(4)]
    cbase = sid * CROWS

    def combine(v_hbm):
        for t in range(5):
            nn = CE if t < 4 else CTAIL
            r0 = cbase + t * CE
            pltpu.sync_copy(acc.at[pl.ds(r0, nn)],
                            rows_v.at[0, pl.ds(0, nn)])
            pltpu.sync_copy(v_hbm.at[pl.ds(r0, nn)],
                            rows_v.at[1, pl.ds(0, nn)])
            pltpu.sync_copy(dinv_hbm.at[pl.ds(r0, nn)],
                            cd_v.at[pl.ds(0, nn)])

            def cb(i, _):
                dv = cd_v[i, pl.ds(0, 16)]
                for c in range(4):
                    a = rows_v[0, i, pl.ds(c * 16, 16)]
                    z = rows_v[1, i, pl.ds(c * 16, 16)]
                    rows_v[2, i, pl.ds(c * 16, 16)] = (a + z) * dv + b2c[c]
                return 0

            lax.fori_loop(0, nn, cb, 0)
            pltpu.sync_copy(
                rows_v.at[2, pl.ds(0, nn)],
                out_hbm.at[pl.ds(r0, nn), pl.ds(cid * DHALF, DHALF)])

    @pl.when(cid == 0)
    def _():
        combine(vl_hbm)

    @pl.when(cid == 1)
    def _():
        combine(vh_hbm)


@functools.partial(
    pl.kernel,
    out_type=jax.ShapeDtypeStruct((N, D_OUT), jnp.float32),
    mesh=_mesh,
    compiler_params=_sc_params,
    scratch_types=[
        pltpu.VMEM((CPW, CE), jnp.int32),
        pltpu.VMEM((CPW, CE), jnp.int32),
        pltpu.VMEM((NBUF2, CE, DHALF), jnp.float32),
        pltpu.VMEM((CE, 16), jnp.float32),
        pltpu.VMEM((8, 16), jnp.float32),
        pltpu.VMEM_SHARED((NP, DHALF), jnp.float32),
        pltpu.SemaphoreType.DMA((NBUF2,)),
        pltpu.SemaphoreType.DMA((NBUF2,)),
    ],
)
def _agg_combine_kernel(vl_hbm, vh_hbm, srcr_hbm, dstr_hbm, dinv_hbm, b2_hbm,
                        out_hbm, src_v, dst_v, rows_v, cd_v, b2_v,
                        acc, gsems, ssems):
    _agg_combine_body(vl_hbm, vh_hbm, srcr_hbm, dstr_hbm, dinv_hbm, b2_hbm,
                      out_hbm, src_v, dst_v, rows_v, cd_v, b2_v,
                      acc, gsems, ssems)


def _dinv_block(degp_ref):
    deg = degp_ref[0, :, 0:1] + degp_ref[1, :, 0:1] + 1.0
    return lax.rsqrt(deg)


def _scale_x_body(degp_ref, x_ref, xsl_ref, xsh_ref):
    xs = x_ref[...] * _dinv_block(degp_ref)
    xsl_ref[...] = xs[:, :DHALF]
    xsh_ref[...] = xs[:, DHALF:]


def _mlp_body(degp_ref, sp_ref, xsl_ref, xsh_ref, w1_ref, b1_ref, w2_ref,
              z2sl_ref, z2sh_ref, dinv16_ref):
    dinv = _dinv_block(degp_ref)
    aggx = jnp.concatenate(
        [sp_ref[0] + xsl_ref[...], sp_ref[1] + xsh_ref[...]], axis=1) * dinv
    h1 = jnp.dot(aggx, w1_ref[...], preferred_element_type=jnp.float32)
    h1 = jnp.maximum(h1 + b1_ref[...], 0.0)
    z2 = jnp.dot(h1, w2_ref[...], preferred_element_type=jnp.float32)
    z2s = z2 * dinv
    z2sl_ref[...] = z2s[:, :DHALF]
    z2sh_ref[...] = z2s[:, DHALF:]
    dinv16_ref[...] = jnp.broadcast_to(dinv, (RB, 16))


def _out_body(degp_ref, sp2_ref, z2sl_ref, z2sh_ref, b2_ref, out_ref):
    dinv = _dinv_block(degp_ref)
    s = jnp.concatenate(
        [sp2_ref[0] + z2sl_ref[...], sp2_ref[1] + z2sh_ref[...]], axis=1)
    out_ref[...] = s * dinv + b2_ref[...]


def _degp_spec():
    return pl.BlockSpec((2, RB, 16), lambda r: (0, r, 0))


def _rows_spec(width):
    return pl.BlockSpec((RB, width), lambda r: (r, 0))


def _part_spec():
    return pl.BlockSpec((2, RB, DHALF), lambda r: (0, r, 0))


def _full_spec(shape):
    nd = len(shape)
    return pl.BlockSpec(shape, lambda r: (0,) * nd)


def _scale_x(degp, x):
    return pl.pallas_call(
        _scale_x_body,
        grid=(ROWS_B,),
        in_specs=[_degp_spec(), _rows_spec(128)],
        out_specs=(_rows_spec(DHALF), _rows_spec(DHALF)),
        out_shape=(jax.ShapeDtypeStruct((N, DHALF), jnp.float32),
                   jax.ShapeDtypeStruct((N, DHALF), jnp.float32)),
    )(degp, x)


def _mlp(degp, sp, xsl, xsh, w1, b1, w2):
    return pl.pallas_call(
        _mlp_body,
        grid=(ROWS_B,),
        in_specs=[_degp_spec(), _part_spec(), _rows_spec(DHALF),
                  _rows_spec(DHALF),
                  _full_spec((D_IN, D_H)), _full_spec((1, D_H)),
                  _full_spec((D_H, D_OUT))],
        out_specs=(_rows_spec(DHALF), _rows_spec(DHALF), _rows_spec(16)),
        out_shape=(jax.ShapeDtypeStruct((N, DHALF), jnp.float32),
                   jax.ShapeDtypeStruct((N, DHALF), jnp.float32),
                   jax.ShapeDtypeStruct((N, 16), jnp.float32)),
    )(degp, sp, xsl, xsh, w1, b1, w2)


def _combine_out(degp, sp2, z2sl, z2sh, b2):
    return pl.pallas_call(
        _out_body,
        grid=(ROWS_B,),
        in_specs=[_degp_spec(), _part_spec(), _rows_spec(DHALF),
                  _rows_spec(DHALF), _full_spec((1, D_OUT))],
        out_specs=_rows_spec(128),
        out_shape=jax.ShapeDtypeStruct((N, D_OUT), jnp.float32),
    )(degp, sp2, z2sl, z2sh, b2)


def kernel(x, edge_index, W1, b1, W2, b2):
    pad = E_PAD - E
    apad = jnp.arange(pad, dtype=jnp.int32)
    src_p = jnp.concatenate([edge_index[0], apad % N])
    dst_p = jnp.concatenate([edge_index[1], N + apad % PAD_ROWS])
    srcr = src_p.reshape(NS, CPW, CE)
    dstr = dst_p.reshape(NS, CPW, CE)

    degp = _deg_kernel(dstr)
    xsl, xsh = _scale_x(degp, x)
    sp = _agg_kernel(xsl, xsh, srcr, dstr)
    z2sl, z2sh, dinv16 = _mlp(degp, sp, xsl, xsh, W1, b1.reshape(1, D_H), W2)
    return _agg_combine_kernel(z2sl, z2sh, srcr, dstr, dinv16,
                               b2.reshape(8, 16))


# speedup vs baseline: 1.0606x; 1.0269x over previous
import functools

import jax
import jax.numpy as jnp
from jax import lax
from jax.experimental import pallas as pl
from jax.experimental.pallas import tpu as pltpu
from jax.experimental.pallas import tpu_sc as plsc

N = 10000
E = 320000
D_IN = 128
D_H = 256
D_OUT = 128
DHALF = 64

NC = 2
NS = 16
CE = 128
CPW = 160
NBUF1 = 6
NBUF2 = 6
E_PAD = NS * CPW * CE
PAD_ROWS = 112
NP = N + PAD_ROWS
RT = NP // NS
ZB = 80

ROWS_B = 10
RB = N // ROWS_B

_mesh = plsc.VectorSubcoreMesh(
    core_axis_name="c", subcore_axis_name="s", num_cores=NC, num_subcores=NS)
_sc_params = pltpu.CompilerParams(use_tc_tiling_on_sc=False)


def _zero_vmem_f32(ref, nrows, ncols):
    zv = jnp.zeros((16,), jnp.float32)

    def body(i, _):
        for c in range(ncols // 16):
            ref[i, pl.ds(c * 16, 16)] = zv
        return 0

    lax.fori_loop(0, nrows, body, 0)


def _deg_body(dstr_hbm, out_hbm, dst_v, ones_v, z16_v, dacc, sem):
    cid = lax.axis_index("c")
    sid = lax.axis_index("s")
    pltpu.async_copy(dstr_hbm.at[sid], dst_v, sem).wait()

    ov = jnp.ones((16,), jnp.float32)

    def fill_ones(i, _):
        ones_v[i, pl.ds(0, 16)] = ov
        return 0

    lax.fori_loop(0, CE, fill_ones, 0)
    _zero_vmem_f32(z16_v, RT, 16)
    base = sid * RT
    pltpu.sync_copy(z16_v, dacc.at[pl.ds(base, RT)])
    plsc.subcore_barrier()

    half = CPW // 2
    lo = cid * half

    def body(j, _):
        pltpu.sync_copy(ones_v, dacc.at[dst_v.at[lo + j]], add=True)
        return 0

    lax.fori_loop(0, half, body, 0)
    plsc.subcore_barrier()
    pltpu.sync_copy(dacc.at[pl.ds(base, RT)], out_hbm.at[cid, pl.ds(base, RT)])


@functools.partial(
    pl.kernel,
    out_type=jax.ShapeDtypeStruct((NC, NP, 16), jnp.float32),
    mesh=_mesh,
    compiler_params=_sc_params,
    scratch_types=[
        pltpu.VMEM((CPW, CE), jnp.int32),
        pltpu.VMEM((CE, 16), jnp.float32),
        pltpu.VMEM((RT, 16), jnp.float32),
        pltpu.VMEM_SHARED((NP, 16), jnp.float32),
        pltpu.SemaphoreType.DMA,
    ],
)
def _deg_kernel(dstr_hbm, out_hbm, dst_v, ones_v, z16_v, dacc, sem):
    _deg_body(dstr_hbm, out_hbm, dst_v, ones_v, z16_v, dacc, sem)


def _zero_acc_slice(rows_v, acc, base):
    zv = jnp.zeros((16,), jnp.float32)

    def zbody(i, _):
        for c in range(DHALF // 16):
            rows_v[0, i, pl.ds(c * 16, 16)] = zv
        return 0

    lax.fori_loop(0, CE, zbody, 0)
    for t in range(4):
        pltpu.sync_copy(rows_v.at[0], acc.at[pl.ds(base + t * CE, CE)])
    pltpu.sync_copy(rows_v.at[0, pl.ds(0, RT - 4 * CE)],
                    acc.at[pl.ds(base + 4 * CE, RT - 4 * CE)])


def _edge_pipeline(v_hbm, src_v, dst_v, rows, acc, gsem, ssem):
    nbuf = len(rows)
    look = nbuf // 2
    groups = CPW // nbuf
    for b in range(look):
        pltpu.async_copy(v_hbm.at[src_v.at[b]], rows[b], gsem[b])

    def slot_static(c):
        b = c % nbuf
        pltpu.make_async_copy(
            v_hbm.at[src_v.at[c]], rows[b], gsem[b]).wait()
        pltpu.async_copy(rows[b], acc.at[dst_v.at[c]], ssem[b], add=True)
        cg = c + look
        bg = (b + look) % nbuf
        if cg < CPW:
            if cg >= nbuf:
                pltpu.make_async_copy(
                    rows[bg], acc.at[dst_v.at[cg - nbuf]], ssem[bg]).wait()
            pltpu.async_copy(v_hbm.at[src_v.at[cg]], rows[bg], gsem[bg])

    def body(j, _):
        for b in range(nbuf):
            c = nbuf * j + b
            pltpu.make_async_copy(
                v_hbm.at[src_v.at[c]], rows[b], gsem[b]).wait()
            pltpu.async_copy(rows[b], acc.at[dst_v.at[c]], ssem[b],
                             add=True)
            cg = c + look
            bg = (b + look) % nbuf

            @pl.when(cg < CPW)
            def _():
                @pl.when(cg >= nbuf)
                def _():
                    pltpu.make_async_copy(
                        rows[bg], acc.at[dst_v.at[cg - nbuf]],
                        ssem[bg]).wait()

                pltpu.async_copy(
                    v_hbm.at[src_v.at[cg]], rows[bg], gsem[bg])

        return 0

    lax.fori_loop(0, groups, body, 0)
    for c in range(groups * nbuf, CPW):
        slot_static(c)
    for c in range(CPW - nbuf, CPW):
        b = c % nbuf
        pltpu.make_async_copy(rows[b], acc.at[dst_v.at[c]], ssem[b]).wait()


def _agg_body(vl_hbm, vh_hbm, srcr_hbm, dstr_hbm, out_hbm,
              src_v, dst_v, rows_v, acc, gsems, ssems):
    nbuf = rows_v.shape[0]
    rows = [rows_v.at[b] for b in range(nbuf)]
    gsem = [gsems.at[b] for b in range(nbuf)]
    ssem = [ssems.at[b] for b in range(nbuf)]
    cid = lax.axis_index("c")
    sid = lax.axis_index("s")
    pltpu.async_copy(srcr_hbm.at[sid], src_v, gsem[0]).wait()
    pltpu.async_copy(dstr_hbm.at[sid], dst_v, gsem[0]).wait()

    base = sid * RT
    _zero_acc_slice(rows_v, acc, base)
    plsc.subcore_barrier()

    @pl.when(cid == 0)
    def _():
        _edge_pipeline(vl_hbm, src_v, dst_v, rows, acc, gsem, ssem)

    @pl.when(cid == 1)
    def _():
        _edge_pipeline(vh_hbm, src_v, dst_v, rows, acc, gsem, ssem)

    plsc.subcore_barrier()
    pltpu.sync_copy(acc.at[pl.ds(base, RT)], out_hbm.at[cid, pl.ds(base, RT)])


@functools.partial(
    pl.kernel,
    out_type=jax.ShapeDtypeStruct((NC, NP, DHALF), jnp.float32),
    mesh=_mesh,
    compiler_params=_sc_params,
    scratch_types=[
        pltpu.VMEM((CPW, CE), jnp.int32),
        pltpu.VMEM((CPW, CE), jnp.int32),
        pltpu.VMEM((NBUF1, CE, DHALF), jnp.float32),
        pltpu.VMEM_SHARED((NP, DHALF), jnp.float32),
        pltpu.SemaphoreType.DMA((NBUF1,)),
        pltpu.SemaphoreType.DMA((NBUF1,)),
    ],
)
def _agg_kernel(vl_hbm, vh_hbm, srcr_hbm, dstr_hbm, out_hbm,
                src_v, dst_v, rows_v, acc, gsems, ssems):
    _agg_body(vl_hbm, vh_hbm, srcr_hbm, dstr_hbm, out_hbm,
              src_v, dst_v, rows_v, acc, gsems, ssems)


CROWS = N // NS
CTAIL = CROWS - 4 * CE

def _agg_combine_body(vl_hbm, vh_hbm, srcr_hbm, dstr_hbm, dinv_hbm, b2_hbm,
                      out_hbm, src_v, dst_v, rows_v, b2_v,
                      acc, gsems, ssems):
    nbuf = rows_v.shape[0]
    rows = [rows_v.at[b] for b in range(nbuf)]
    gsem = [gsems.at[b] for b in range(nbuf)]
    ssem = [ssems.at[b] for b in range(nbuf)]
    cid = lax.axis_index("c")
    sid = lax.axis_index("s")
    pltpu.async_copy(srcr_hbm.at[sid], src_v, gsem[0]).wait()
    pltpu.async_copy(dstr_hbm.at[sid], dst_v, gsem[0]).wait()
    pltpu.async_copy(b2_hbm, b2_v, gsem[1]).wait()

    base = sid * RT
    _zero_acc_slice(rows_v, acc, base)
    plsc.subcore_barrier()

    @pl.when(cid == 0)
    def _():
        _edge_pipeline(vl_hbm, src_v, dst_v, rows, acc, gsem, ssem)

    @pl.when(cid == 1)
    def _():
        _edge_pipeline(vh_hbm, src_v, dst_v, rows, acc, gsem, ssem)

    plsc.subcore_barrier()

    b2c = [b2_v[cid * 4 + c, pl.ds(0, 16)] for c in range(4)]
    cbase = sid * CROWS

    def combine(v_hbm):
        for t in range(5):
            nn = CE if t < 4 else CTAIL
            r0 = cbase + t * CE
            pltpu.sync_copy(acc.at[pl.ds(r0, nn)],
                            rows_v.at[0, pl.ds(0, nn)])
            pltpu.sync_copy(v_hbm.at[pl.ds(r0, nn)],
                            rows_v.at[1, pl.ds(0, nn)])
            pltpu.sync_copy(dinv_hbm.at[pl.ds(r0, nn)],
                            rows_v.at[3, pl.ds(0, nn), pl.ds(0, 16)])

            def cb(i, _):
                dv = rows_v[3, i, pl.ds(0, 16)]
                for c in range(4):
                    a = rows_v[0, i, pl.ds(c * 16, 16)]
                    z = rows_v[1, i, pl.ds(c * 16, 16)]
                    rows_v[2, i, pl.ds(c * 16, 16)] = (a + z) * dv + b2c[c]
                return 0

            lax.fori_loop(0, nn, cb, 0)
            pltpu.sync_copy(
                rows_v.at[2, pl.ds(0, nn)],
                out_hbm.at[pl.ds(r0, nn), pl.ds(cid * DHALF, DHALF)])

    @pl.when(cid == 0)
    def _():
        combine(vl_hbm)

    @pl.when(cid == 1)
    def _():
        combine(vh_hbm)


@functools.partial(
    pl.kernel,
    out_type=jax.ShapeDtypeStruct((N, D_OUT), jnp.float32),
    mesh=_mesh,
    compiler_params=_sc_params,
    scratch_types=[
        pltpu.VMEM((CPW, CE), jnp.int32),
        pltpu.VMEM((CPW, CE), jnp.int32),
        pltpu.VMEM((NBUF2, CE, DHALF), jnp.float32),
        pltpu.VMEM((8, 16), jnp.float32),
        pltpu.VMEM_SHARED((NP, DHALF), jnp.float32),
        pltpu.SemaphoreType.DMA((NBUF2,)),
        pltpu.SemaphoreType.DMA((NBUF2,)),
    ],
)
def _agg_combine_kernel(vl_hbm, vh_hbm, srcr_hbm, dstr_hbm, dinv_hbm, b2_hbm,
                        out_hbm, src_v, dst_v, rows_v, b2_v,
                        acc, gsems, ssems):
    _agg_combine_body(vl_hbm, vh_hbm, srcr_hbm, dstr_hbm, dinv_hbm, b2_hbm,
                      out_hbm, src_v, dst_v, rows_v, b2_v,
                      acc, gsems, ssems)


def _dinv_block(degp_ref):
    deg = degp_ref[0, :, 0:1] + degp_ref[1, :, 0:1] + 1.0
    return lax.rsqrt(deg)


def _scale_x_body(degp_ref, x_ref, xsl_ref, xsh_ref):
    xs = x_ref[...] * _dinv_block(degp_ref)
    xsl_ref[...] = xs[:, :DHALF]
    xsh_ref[...] = xs[:, DHALF:]


def _mlp_body(degp_ref, sp_ref, xsl_ref, xsh_ref, w1_ref, b1_ref, w2_ref,
              z2sl_ref, z2sh_ref, dinv16_ref):
    dinv = _dinv_block(degp_ref)
    aggx = jnp.concatenate(
        [sp_ref[0] + xsl_ref[...], sp_ref[1] + xsh_ref[...]], axis=1) * dinv
    h1 = jnp.dot(aggx, w1_ref[...], preferred_element_type=jnp.float32)
    h1 = jnp.maximum(h1 + b1_ref[...], 0.0)
    z2 = jnp.dot(h1, w2_ref[...], preferred_element_type=jnp.float32)
    z2s = z2 * dinv
    z2sl_ref[...] = z2s[:, :DHALF]
    z2sh_ref[...] = z2s[:, DHALF:]
    dinv16_ref[...] = jnp.broadcast_to(dinv, (RB, 16))


def _out_body(degp_ref, sp2_ref, z2sl_ref, z2sh_ref, b2_ref, out_ref):
    dinv = _dinv_block(degp_ref)
    s = jnp.concatenate(
        [sp2_ref[0] + z2sl_ref[...], sp2_ref[1] + z2sh_ref[...]], axis=1)
    out_ref[...] = s * dinv + b2_ref[...]


def _degp_spec():
    return pl.BlockSpec((2, RB, 16), lambda r: (0, r, 0))


def _rows_spec(width):
    return pl.BlockSpec((RB, width), lambda r: (r, 0))


def _part_spec():
    return pl.BlockSpec((2, RB, DHALF), lambda r: (0, r, 0))


def _full_spec(shape):
    nd = len(shape)
    return pl.BlockSpec(shape, lambda r: (0,) * nd)


def _scale_x(degp, x):
    return pl.pallas_call(
        _scale_x_body,
        grid=(ROWS_B,),
        in_specs=[_degp_spec(), _rows_spec(128)],
        out_specs=(_rows_spec(DHALF), _rows_spec(DHALF)),
        out_shape=(jax.ShapeDtypeStruct((N, DHALF), jnp.float32),
                   jax.ShapeDtypeStruct((N, DHALF), jnp.float32)),
    )(degp, x)


def _mlp(degp, sp, xsl, xsh, w1, b1, w2):
    return pl.pallas_call(
        _mlp_body,
        grid=(ROWS_B,),
        in_specs=[_degp_spec(), _part_spec(), _rows_spec(DHALF),
                  _rows_spec(DHALF),
                  _full_spec((D_IN, D_H)), _full_spec((1, D_H)),
                  _full_spec((D_H, D_OUT))],
        out_specs=(_rows_spec(DHALF), _rows_spec(DHALF), _rows_spec(16)),
        out_shape=(jax.ShapeDtypeStruct((N, DHALF), jnp.float32),
                   jax.ShapeDtypeStruct((N, DHALF), jnp.float32),
                   jax.ShapeDtypeStruct((N, 16), jnp.float32)),
    )(degp, sp, xsl, xsh, w1, b1, w2)


def _combine_out(degp, sp2, z2sl, z2sh, b2):
    return pl.pallas_call(
        _out_body,
        grid=(ROWS_B,),
        in_specs=[_degp_spec(), _part_spec(), _rows_spec(DHALF),
                  _rows_spec(DHALF), _full_spec((1, D_OUT))],
        out_specs=_rows_spec(128),
        out_shape=jax.ShapeDtypeStruct((N, D_OUT), jnp.float32),
    )(degp, sp2, z2sl, z2sh, b2)


def kernel(x, edge_index, W1, b1, W2, b2):
    pad = E_PAD - E
    apad = jnp.arange(pad, dtype=jnp.int32)
    src_p = jnp.concatenate([edge_index[0], apad % N])
    dst_p = jnp.concatenate([edge_index[1], N + apad % PAD_ROWS])
    srcr = src_p.reshape(NS, CPW, CE)
    dstr = dst_p.reshape(NS, CPW, CE)

    degp = _deg_kernel(dstr)
    xsl, xsh = _scale_x(degp, x)
    sp = _agg_kernel(xsl, xsh, srcr, dstr)
    z2sl, z2sh, dinv16 = _mlp(degp, sp, xsl, xsh, W1, b1.reshape(1, D_H), W2)
    return _agg_combine_kernel(z2sl, z2sh, srcr, dstr, dinv16,
                               b2.reshape(8, 16))
